# baseline (speedup 1.0000x reference)
import jax
import jax.numpy as jnp
from jax import lax
from jax.experimental import pallas as pl
from jax.experimental.pallas import tpu as pltpu
from jax.experimental.pallas import tpu_sc as plsc

SEQ_LEN = 200
BATCH = 4096
EMBED_DIM = 64
OUTPUT_DIM = 2
LANES = 16
NUM_CORES = 2
NUM_SUBCORES = 16
NUM_WORKERS = NUM_CORES * NUM_SUBCORES
B_PER_W = BATCH // NUM_WORKERS
CHUNKS = EMBED_DIM // LANES


def _sc_body(text_ref, table_ref, out_ref, idx_v, rows0, rows1, acc, sem0, sem1):
  wid = lax.axis_index("s") * NUM_CORES + lax.axis_index("c")
  base = wid * B_PER_W

  pltpu.sync_copy(text_ref.at[:, pl.ds(base, B_PER_W)], idx_v)

  def start(t, buf, sem):
    pltpu.async_copy(table_ref.at[idx_v.at[t]], buf, sem)

  def wait(buf, sem):
    pltpu.make_async_copy(table_ref.at[idx_v.at[0]], buf, sem).wait()

  def accum(buf, first=False):
    @pl.loop(0, B_PER_W, unroll=2)
    def _(b):
      for c in range(CHUNKS):
        x = buf[b, pl.ds(c * LANES, LANES)]
        if first:
          acc[b, pl.ds(c * LANES, LANES)] = x
        else:
          plsc.addupdate(acc.at[b, pl.ds(c * LANES, LANES)], x)

  start(0, rows0, sem0)
  start(1, rows1, sem1)
  wait(rows0, sem0)
  accum(rows0, first=True)
  start(2, rows0, sem0)
  wait(rows1, sem1)
  accum(rows1)
  start(3, rows1, sem1)

  @pl.loop(1, SEQ_LEN // 2 - 1)
  def _(g):
    wait(rows0, sem0)
    accum(rows0)
    start(2 * g + 2, rows0, sem0)
    wait(rows1, sem1)
    accum(rows1)
    start(2 * g + 3, rows1, sem1)

  wait(rows0, sem0)
  accum(rows0)
  wait(rows1, sem1)
  accum(rows1)

  pltpu.sync_copy(acc, out_ref.at[pl.ds(base, B_PER_W)])


def _sc_embed_bag(text, table):
  mesh = plsc.VectorSubcoreMesh(core_axis_name="c", subcore_axis_name="s")
  return pl.kernel(
      _sc_body,
      out_type=jax.ShapeDtypeStruct((BATCH, EMBED_DIM), jnp.float32),
      mesh=mesh,
      scratch_types=[
          pltpu.VMEM((SEQ_LEN, B_PER_W), jnp.int32),
          pltpu.VMEM((B_PER_W, EMBED_DIM), jnp.float32),
          pltpu.VMEM((B_PER_W, EMBED_DIM), jnp.float32),
          pltpu.VMEM((B_PER_W, EMBED_DIM), jnp.float32),
          pltpu.SemaphoreType.DMA,
          pltpu.SemaphoreType.DMA,
      ],
      compiler_params=pltpu.CompilerParams(use_tc_tiling_on_sc=False),
  )(text, table)


def _fc_body(x_ref, w_ref, b_ref, o_ref):
  x = x_ref[...] * (1.0 / SEQ_LEN)
  o_ref[...] = (
      lax.dot_general(
          x, w_ref[...], (((1,), (1,)), ((), ())),
          preferred_element_type=jnp.float32,
      )
      + b_ref[...]
  )


def _fc(pooled_sum, fc_w, fc_b):
  blk = 512
  return pl.pallas_call(
      _fc_body,
      grid=(BATCH // blk,),
      in_specs=[
          pl.BlockSpec((blk, EMBED_DIM), lambda i: (i, 0)),
          pl.BlockSpec((OUTPUT_DIM, EMBED_DIM), lambda i: (0, 0)),
          pl.BlockSpec((1, OUTPUT_DIM), lambda i: (0, 0)),
      ],
      out_specs=pl.BlockSpec((blk, OUTPUT_DIM), lambda i: (i, 0)),
      out_shape=jax.ShapeDtypeStruct((BATCH, OUTPUT_DIM), jnp.float32),
  )(pooled_sum, fc_w, fc_b)


@jax.jit
def kernel(text, embed_table, fc_w, fc_b):
  text = text.astype(jnp.int32)
  pooled_sum = _sc_embed_bag(text, embed_table)
  return _fc(pooled_sum, fc_w, jnp.reshape(fc_b, (1, OUTPUT_DIM)))
